# Initial kernel scaffold; baseline (speedup 1.0000x reference)
import jax
import jax.numpy as jnp
from jax.experimental import pallas as pl


def kernel(x, edge_index, enc_W0, enc_b0, enc_W1, enc_b1, dec_W0, dec_b0, dec_W1, dec_b1):
    raise NotImplementedError("write your pallas kernel here")


# speedup vs baseline: 17.6233x; 17.6233x over previous
import functools

import jax
import jax.numpy as jnp
from jax import lax
from jax.experimental import pallas as pl
from jax.experimental.pallas import tpu as pltpu
from jax.experimental.pallas import tpu_sc as plsc

_N = 10000
_E = 320000
_DP = 128
_DO = 128
_NTILES = 32
_EPT = _E // _NTILES
_CH = 125
_NCH = _EPT // _CH
_NPAD = 10240
_RPT = _NPAD // 16


def _flat_tile_id():
    c = lax.axis_index("c")
    s = lax.axis_index("s")
    return c, s, c * 16 + s


@functools.cache
def _make_edge_kernel(d):
    mesh = plsc.VectorSubcoreMesh(core_axis_name="c", subcore_axis_name="s")

    @functools.partial(
        pl.kernel,
        out_type=jax.ShapeDtypeStruct((2, _NPAD, d), jnp.float32),
        mesh=mesh,
        scratch_types=[
            pltpu.VMEM((_NCH, _CH), jnp.int32),
            pltpu.VMEM((_NCH, _CH), jnp.int32),
            pltpu.VMEM((_CH, d), jnp.float32),
            pltpu.VMEM_SHARED((_NPAD, d), jnp.float32),
            pltpu.SemaphoreType.DMA,
        ],
    )
    def edge_kernel(z_hbm, src_hbm, dst_hbm, zero_hbm, out_hbm,
                    src_v, dst_v, rows_v, acc_sh, sem):
        c, s, w = _flat_tile_id()
        row0 = pl.multiple_of(s * _RPT, 8)
        pltpu.sync_copy(src_hbm.at[w], src_v)
        pltpu.sync_copy(dst_hbm.at[w], dst_v)
        pltpu.sync_copy(zero_hbm.at[pl.ds(row0, _RPT)],
                        acc_sh.at[pl.ds(row0, _RPT)])
        plsc.subcore_barrier()

        def body(j, carry):
            pltpu.async_copy(z_hbm.at[src_v.at[j]], rows_v, sem).wait()
            pltpu.sync_copy(rows_v, acc_sh.at[dst_v.at[j]], add=True)
            return carry

        lax.fori_loop(0, _NCH, body, 0)
        plsc.subcore_barrier()
        pltpu.sync_copy(acc_sh.at[pl.ds(row0, _RPT)],
                        out_hbm.at[c, pl.ds(row0, _RPT)])

    return edge_kernel


@functools.cache
def _make_deg_kernel():
    mesh = plsc.VectorSubcoreMesh(core_axis_name="c", subcore_axis_name="s")

    @functools.partial(
        pl.kernel,
        out_type=jax.ShapeDtypeStruct((2, _NPAD, 128), jnp.float32),
        mesh=mesh,
        scratch_types=[
            pltpu.VMEM((_NCH, _CH), jnp.int32),
            pltpu.VMEM((_CH, 128), jnp.float32),
            pltpu.VMEM_SHARED((_NPAD, 128), jnp.float32),
        ],
    )
    def deg_kernel(dst_hbm, ones_hbm, zero_hbm, out_hbm, dst_v, ones_v, acc_sh):
        c, s, w = _flat_tile_id()
        row0 = pl.multiple_of(s * _RPT, 8)
        pltpu.sync_copy(dst_hbm.at[w], dst_v)
        pltpu.sync_copy(ones_hbm, ones_v)
        pltpu.sync_copy(zero_hbm.at[pl.ds(row0, _RPT)],
                        acc_sh.at[pl.ds(row0, _RPT)])
        plsc.subcore_barrier()

        def body(j, carry):
            pltpu.sync_copy(ones_v, acc_sh.at[dst_v.at[j]], add=True)
            return carry

        lax.fori_loop(0, _NCH, body, 0)
        plsc.subcore_barrier()
        pltpu.sync_copy(acc_sh.at[pl.ds(row0, _RPT)],
                        out_hbm.at[c, pl.ds(row0, _RPT)])

    return deg_kernel


_TCR = 1000


def _dinv_block(degp_ref):
    deg = degp_ref[0, :, 0:1] + degp_ref[1, :, 0:1] + 1.0
    return lax.rsqrt(deg)


def _tc_first(degp, x, w0):
    din, dout = x.shape[1], w0.shape[1]

    def body(degp_ref, x_ref, w_ref, z_ref):
        dinv = _dinv_block(degp_ref)
        z_ref[...] = dinv * jnp.dot(x_ref[...], w_ref[...],
                                    preferred_element_type=jnp.float32)

    return pl.pallas_call(
        body,
        grid=(_N // _TCR,),
        in_specs=[
            pl.BlockSpec((2, _TCR, 128), lambda i: (0, i, 0)),
            pl.BlockSpec((_TCR, din), lambda i: (i, 0)),
            pl.BlockSpec((din, dout), lambda i: (0, 0)),
        ],
        out_specs=pl.BlockSpec((_TCR, dout), lambda i: (i, 0)),
        out_shape=jax.ShapeDtypeStruct((_N, dout), jnp.float32),
    )(degp, x, w0)


def _tc_mid(degp, u, zp, b, w):
    din, dout = w.shape

    def body(degp_ref, u_ref, zp_ref, b_ref, w_ref, z_ref):
        dinv = _dinv_block(degp_ref)
        h = jnp.maximum(dinv * (u_ref[0] + u_ref[1] + zp_ref[...]) + b_ref[...], 0.0)
        z_ref[...] = dinv * jnp.dot(h, w_ref[...],
                                    preferred_element_type=jnp.float32)

    return pl.pallas_call(
        body,
        grid=(_N // _TCR,),
        in_specs=[
            pl.BlockSpec((2, _TCR, 128), lambda i: (0, i, 0)),
            pl.BlockSpec((2, _TCR, din), lambda i: (0, i, 0)),
            pl.BlockSpec((_TCR, din), lambda i: (i, 0)),
            pl.BlockSpec((1, din), lambda i: (0, 0)),
            pl.BlockSpec((din, dout), lambda i: (0, 0)),
        ],
        out_specs=pl.BlockSpec((_TCR, dout), lambda i: (i, 0)),
        out_shape=jax.ShapeDtypeStruct((_N, dout), jnp.float32),
    )(degp, u, zp, b, w)


def _tc_last(degp, u, zp, b):
    dout = zp.shape[1]

    def body(degp_ref, u_ref, zp_ref, b_ref, o_ref):
        dinv = _dinv_block(degp_ref)
        o_ref[...] = jnp.maximum(
            dinv * (u_ref[0] + u_ref[1] + zp_ref[...]) + b_ref[...], 0.0)

    return pl.pallas_call(
        body,
        grid=(_N // _TCR,),
        in_specs=[
            pl.BlockSpec((2, _TCR, 128), lambda i: (0, i, 0)),
            pl.BlockSpec((2, _TCR, dout), lambda i: (0, i, 0)),
            pl.BlockSpec((_TCR, dout), lambda i: (i, 0)),
            pl.BlockSpec((1, dout), lambda i: (0, 0)),
        ],
        out_specs=pl.BlockSpec((_TCR, dout), lambda i: (i, 0)),
        out_shape=jax.ShapeDtypeStruct((_N, dout), jnp.float32),
    )(degp, u, zp, b)


def _pad_w(w, rows, cols):
    return jnp.pad(w, ((0, rows - w.shape[0]), (0, cols - w.shape[1])))


def _pad_b(b, cols):
    return jnp.pad(b, (0, cols - b.shape[0])).reshape(1, cols)


def kernel(x, edge_index, enc_W0, enc_b0, enc_W1, enc_b1,
           dec_W0, dec_b0, dec_W1, dec_b1):
    src = edge_index[0].reshape(_NTILES, _NCH, _CH)
    dst = edge_index[1].reshape(_NTILES, _NCH, _CH)

    w0 = _pad_w(enc_W0, 128, _DP)
    w1 = _pad_w(enc_W1, _DP, _DP)
    w2 = _pad_w(dec_W0, _DP, _DP)
    w3 = _pad_w(dec_W1, _DP, _DO)
    b0 = _pad_b(enc_b0, _DP)
    b1 = _pad_b(enc_b1, _DP)
    b2 = _pad_b(dec_b0, _DP)
    b3 = _pad_b(dec_b1, _DO)

    zeros_dp = jnp.zeros((_NPAD, _DP), jnp.float32)
    zeros_do = jnp.zeros((_NPAD, _DO), jnp.float32)
    ones128 = jnp.ones((_CH, 128), jnp.float32)

    degp = _make_deg_kernel()(dst, ones128, zeros_dp)

    edge_dp = _make_edge_kernel(_DP)
    edge_do = _make_edge_kernel(_DO)

    z0 = _tc_first(degp, x, w0)
    u0 = edge_dp(z0, src, dst, zeros_dp)
    z1 = _tc_mid(degp, u0, z0, b0, w1)
    u1 = edge_dp(z1, src, dst, zeros_dp)
    z2 = _tc_mid(degp, u1, z1, b1, w2)
    u2 = edge_dp(z2, src, dst, zeros_dp)
    z3 = _tc_mid(degp, u2, z2, b2, w3)
    u3 = edge_do(z3, src, dst, zeros_do)
    return _tc_last(degp, u3, z3, b3)


# speedup vs baseline: 21.9743x; 1.2469x over previous
import functools

import jax
import jax.numpy as jnp
from jax import lax
from jax.experimental import pallas as pl
from jax.experimental.pallas import tpu as pltpu
from jax.experimental.pallas import tpu_sc as plsc

_N = 10000
_E = 320000
_DP = 128
_DO = 128
_NTILES = 32
_EPT = _E // _NTILES
_CH = 125
_NCH = _EPT // _CH
_NPAD = 10240
_RPT = _NPAD // 16


def _flat_tile_id():
    c = lax.axis_index("c")
    s = lax.axis_index("s")
    return c, s, c * 16 + s


@functools.cache
def _make_edge_kernel(d):
    mesh = plsc.VectorSubcoreMesh(core_axis_name="c", subcore_axis_name="s")

    nbuf = 2
    hch = _NCH // 2

    @functools.partial(
        pl.kernel,
        out_type=jax.ShapeDtypeStruct((2, _NPAD, d), jnp.float32),
        mesh=mesh,
        scratch_types=[
            pltpu.VMEM((hch, _CH), jnp.int32),
            pltpu.VMEM((hch, _CH), jnp.int32),
            pltpu.VMEM((nbuf, _CH, d), jnp.float32),
            pltpu.VMEM_SHARED((_NPAD, d), jnp.float32),
            pltpu.SemaphoreType.DMA,
        ],
    )
    def edge_kernel(z_hbm, src_hbm, dst_hbm, zero_hbm, out_hbm,
                    src_v, dst_v, rows_v, acc_sh, sem):
        c, s, w = _flat_tile_id()
        row0 = pl.multiple_of(s * _RPT, 8)
        pltpu.sync_copy(zero_hbm.at[pl.ds(row0, _RPT)],
                        acc_sh.at[pl.ds(row0, _RPT)])
        plsc.subcore_barrier()

        def gather(j, b):
            pltpu.async_copy(z_hbm.at[src_v.at[j]], rows_v.at[b], sem)

        def gather_wait(j, b):
            pltpu.make_async_copy(z_hbm.at[src_v.at[j]], rows_v.at[b],
                                  sem).wait()

        def scatter(j, b):
            pltpu.sync_copy(rows_v.at[b], acc_sh.at[dst_v.at[j]], add=True)

        for h in range(2):
            pltpu.sync_copy(src_hbm.at[w, h], src_v)
            pltpu.sync_copy(dst_hbm.at[w, h], dst_v)
            gather(0, 0)

            def body(t, carry):
                j = 2 * t
                gather_wait(j, 0)
                gather(j + 1, 1)
                scatter(j, 0)
                gather_wait(j + 1, 1)

                @pl.when(j + 2 < hch)
                def _():
                    gather(j + 2, 0)

                scatter(j + 1, 1)
                return carry

            lax.fori_loop(0, hch // 2, body, 0)

        plsc.subcore_barrier()
        pltpu.sync_copy(acc_sh.at[pl.ds(row0, _RPT)],
                        out_hbm.at[c, pl.ds(row0, _RPT)])

    return edge_kernel


@functools.cache
def _make_deg_kernel():
    mesh = plsc.VectorSubcoreMesh(core_axis_name="c", subcore_axis_name="s")

    @functools.partial(
        pl.kernel,
        out_type=jax.ShapeDtypeStruct((2, _NPAD, 128), jnp.float32),
        mesh=mesh,
        scratch_types=[
            pltpu.VMEM((_NCH // 2, _CH), jnp.int32),
            pltpu.VMEM((_CH, 128), jnp.float32),
            pltpu.VMEM_SHARED((_NPAD, 128), jnp.float32),
        ],
    )
    def deg_kernel(dst_hbm, ones_hbm, zero_hbm, out_hbm, dst_v, ones_v, acc_sh):
        c, s, w = _flat_tile_id()
        row0 = pl.multiple_of(s * _RPT, 8)
        pltpu.sync_copy(ones_hbm, ones_v)
        pltpu.sync_copy(zero_hbm.at[pl.ds(row0, _RPT)],
                        acc_sh.at[pl.ds(row0, _RPT)])
        plsc.subcore_barrier()

        def body(j, carry):
            pltpu.sync_copy(ones_v, acc_sh.at[dst_v.at[j]], add=True)
            return carry

        for h in range(2):
            pltpu.sync_copy(dst_hbm.at[w, h], dst_v)
            lax.fori_loop(0, _NCH // 2, body, 0)
        plsc.subcore_barrier()
        pltpu.sync_copy(acc_sh.at[pl.ds(row0, _RPT)],
                        out_hbm.at[c, pl.ds(row0, _RPT)])

    return deg_kernel


_TCR = 1000


def _dinv_block(degp_ref):
    deg = degp_ref[0, :, 0:1] + degp_ref[1, :, 0:1] + 1.0
    return lax.rsqrt(deg)


def _tc_first(degp, x, w0):
    din, dout = x.shape[1], w0.shape[1]

    def body(degp_ref, x_ref, w_ref, z_ref):
        dinv = _dinv_block(degp_ref)
        z_ref[...] = dinv * jnp.dot(x_ref[...], w_ref[...],
                                    preferred_element_type=jnp.float32)

    return pl.pallas_call(
        body,
        grid=(_N // _TCR,),
        in_specs=[
            pl.BlockSpec((2, _TCR, 128), lambda i: (0, i, 0)),
            pl.BlockSpec((_TCR, din), lambda i: (i, 0)),
            pl.BlockSpec((din, dout), lambda i: (0, 0)),
        ],
        out_specs=pl.BlockSpec((_TCR, dout), lambda i: (i, 0)),
        out_shape=jax.ShapeDtypeStruct((_N, dout), jnp.float32),
    )(degp, x, w0)


def _tc_mid(degp, u, zp, b, w):
    din, dout = w.shape

    def body(degp_ref, u_ref, zp_ref, b_ref, w_ref, z_ref):
        dinv = _dinv_block(degp_ref)
        h = jnp.maximum(dinv * (u_ref[0] + u_ref[1] + zp_ref[...]) + b_ref[...], 0.0)
        z_ref[...] = dinv * jnp.dot(h, w_ref[...],
                                    preferred_element_type=jnp.float32)

    return pl.pallas_call(
        body,
        grid=(_N // _TCR,),
        in_specs=[
            pl.BlockSpec((2, _TCR, 128), lambda i: (0, i, 0)),
            pl.BlockSpec((2, _TCR, din), lambda i: (0, i, 0)),
            pl.BlockSpec((_TCR, din), lambda i: (i, 0)),
            pl.BlockSpec((1, din), lambda i: (0, 0)),
            pl.BlockSpec((din, dout), lambda i: (0, 0)),
        ],
        out_specs=pl.BlockSpec((_TCR, dout), lambda i: (i, 0)),
        out_shape=jax.ShapeDtypeStruct((_N, dout), jnp.float32),
    )(degp, u, zp, b, w)


def _tc_last(degp, u, zp, b):
    dout = zp.shape[1]

    def body(degp_ref, u_ref, zp_ref, b_ref, o_ref):
        dinv = _dinv_block(degp_ref)
        o_ref[...] = jnp.maximum(
            dinv * (u_ref[0] + u_ref[1] + zp_ref[...]) + b_ref[...], 0.0)

    return pl.pallas_call(
        body,
        grid=(_N // _TCR,),
        in_specs=[
            pl.BlockSpec((2, _TCR, 128), lambda i: (0, i, 0)),
            pl.BlockSpec((2, _TCR, dout), lambda i: (0, i, 0)),
            pl.BlockSpec((_TCR, dout), lambda i: (i, 0)),
            pl.BlockSpec((1, dout), lambda i: (0, 0)),
        ],
        out_specs=pl.BlockSpec((_TCR, dout), lambda i: (i, 0)),
        out_shape=jax.ShapeDtypeStruct((_N, dout), jnp.float32),
    )(degp, u, zp, b)


def _pad_w(w, rows, cols):
    return jnp.pad(w, ((0, rows - w.shape[0]), (0, cols - w.shape[1])))


def _pad_b(b, cols):
    return jnp.pad(b, (0, cols - b.shape[0])).reshape(1, cols)


def kernel(x, edge_index, enc_W0, enc_b0, enc_W1, enc_b1,
           dec_W0, dec_b0, dec_W1, dec_b1):
    src = edge_index[0].reshape(_NTILES, 2, _NCH // 2, _CH)
    dst = edge_index[1].reshape(_NTILES, 2, _NCH // 2, _CH)

    w0 = _pad_w(enc_W0, 128, _DP)
    w1 = _pad_w(enc_W1, _DP, _DP)
    w2 = _pad_w(dec_W0, _DP, _DP)
    w3 = _pad_w(dec_W1, _DP, _DO)
    b0 = _pad_b(enc_b0, _DP)
    b1 = _pad_b(enc_b1, _DP)
    b2 = _pad_b(dec_b0, _DP)
    b3 = _pad_b(dec_b1, _DO)

    zeros_dp = jnp.zeros((_NPAD, _DP), jnp.float32)
    zeros_do = jnp.zeros((_NPAD, _DO), jnp.float32)
    ones128 = jnp.ones((_CH, 128), jnp.float32)

    degp = _make_deg_kernel()(dst, ones128, zeros_dp)

    edge_dp = _make_edge_kernel(_DP)
    edge_do = _make_edge_kernel(_DO)

    z0 = _tc_first(degp, x, w0)
    u0 = edge_dp(z0, src, dst, zeros_dp)
    z1 = _tc_mid(degp, u0, z0, b0, w1)
    u1 = edge_dp(z1, src, dst, zeros_dp)
    z2 = _tc_mid(degp, u1, z1, b1, w2)
    u2 = edge_dp(z2, src, dst, zeros_dp)
    z3 = _tc_mid(degp, u2, z2, b2, w3)
    u3 = edge_do(z3, src, dst, zeros_do)
    return _tc_last(degp, u3, z3, b3)


# speedup vs baseline: 25.0392x; 1.1395x over previous
import functools

import jax
import jax.numpy as jnp
from jax import lax
from jax.experimental import pallas as pl
from jax.experimental.pallas import tpu as pltpu
from jax.experimental.pallas import tpu_sc as plsc

_N = 10000
_E = 320000
_DP = 128
_DO = 128
_NTILES = 32
_EPT = _E // _NTILES
_CH = 125
_NCH = _EPT // _CH
_NPAD = 10240
_RPT = _NPAD // 16


def _flat_tile_id():
    c = lax.axis_index("c")
    s = lax.axis_index("s")
    return c, s, c * 16 + s


@functools.cache
def _make_edge_kernel(d):
    mesh = plsc.VectorSubcoreMesh(core_axis_name="c", subcore_axis_name="s")

    nbuf = 2
    hch = _NCH // 2

    @functools.partial(
        pl.kernel,
        out_type=jax.ShapeDtypeStruct((2, _NPAD, d), jnp.float32),
        mesh=mesh,
        scratch_types=[
            pltpu.VMEM((hch, _CH), jnp.int32),
            pltpu.VMEM((hch, _CH), jnp.int32),
            pltpu.VMEM((nbuf, _CH, d), jnp.float32),
            pltpu.VMEM_SHARED((_NPAD, d), jnp.float32),
            pltpu.SemaphoreType.DMA,
            pltpu.SemaphoreType.DMA,
        ],
    )
    def edge_kernel(z_hbm, src_hbm, dst_hbm, zero_hbm, out_hbm,
                    src_v, dst_v, rows_v, acc_sh, sem0, sem1):
        sems = (sem0, sem1)
        c, s, w = _flat_tile_id()
        row0 = pl.multiple_of(s * _RPT, 8)
        pltpu.sync_copy(zero_hbm.at[pl.ds(row0, _RPT)],
                        acc_sh.at[pl.ds(row0, _RPT)])
        plsc.subcore_barrier()

        def gather(j, b):
            pltpu.async_copy(z_hbm.at[src_v.at[j]], rows_v.at[b], sems[b])

        def gather_wait(j, b):
            pltpu.make_async_copy(z_hbm.at[src_v.at[j]], rows_v.at[b],
                                  sems[b]).wait()

        def scatter(j, b):
            pltpu.sync_copy(rows_v.at[b], acc_sh.at[dst_v.at[j]], add=True)

        for h in range(2):
            pltpu.sync_copy(src_hbm.at[w, h], src_v)
            pltpu.sync_copy(dst_hbm.at[w, h], dst_v)
            gather(0, 0)
            gather(1, 1)

            def body(t, carry):
                j = 2 * t
                gather_wait(j, 0)
                scatter(j, 0)

                @pl.when(j + 2 < hch)
                def _():
                    gather(j + 2, 0)

                gather_wait(j + 1, 1)
                scatter(j + 1, 1)

                @pl.when(j + 3 < hch)
                def _():
                    gather(j + 3, 1)

                return carry

            lax.fori_loop(0, hch // 2, body, 0)

        plsc.subcore_barrier()
        pltpu.sync_copy(acc_sh.at[pl.ds(row0, _RPT)],
                        out_hbm.at[c, pl.ds(row0, _RPT)])

    return edge_kernel


@functools.cache
def _make_deg_kernel():
    mesh = plsc.VectorSubcoreMesh(core_axis_name="c", subcore_axis_name="s")

    @functools.partial(
        pl.kernel,
        out_type=jax.ShapeDtypeStruct((2, _NPAD, 128), jnp.float32),
        mesh=mesh,
        scratch_types=[
            pltpu.VMEM((_NCH // 2, _CH), jnp.int32),
            pltpu.VMEM((_CH, 128), jnp.float32),
            pltpu.VMEM_SHARED((_NPAD, 128), jnp.float32),
        ],
    )
    def deg_kernel(dst_hbm, ones_hbm, zero_hbm, out_hbm, dst_v, ones_v, acc_sh):
        c, s, w = _flat_tile_id()
        row0 = pl.multiple_of(s * _RPT, 8)
        pltpu.sync_copy(ones_hbm, ones_v)
        pltpu.sync_copy(zero_hbm.at[pl.ds(row0, _RPT)],
                        acc_sh.at[pl.ds(row0, _RPT)])
        plsc.subcore_barrier()

        def body(j, carry):
            pltpu.sync_copy(ones_v, acc_sh.at[dst_v.at[j]], add=True)
            return carry

        for h in range(2):
            pltpu.sync_copy(dst_hbm.at[w, h], dst_v)
            lax.fori_loop(0, _NCH // 2, body, 0)
        plsc.subcore_barrier()
        pltpu.sync_copy(acc_sh.at[pl.ds(row0, _RPT)],
                        out_hbm.at[c, pl.ds(row0, _RPT)])

    return deg_kernel


_TCR = 1000


def _dinv_block(degp_ref):
    deg = degp_ref[0, :, 0:1] + degp_ref[1, :, 0:1] + 1.0
    return lax.rsqrt(deg)


def _tc_first(degp, x, w0):
    din, dout = x.shape[1], w0.shape[1]

    def body(degp_ref, x_ref, w_ref, z_ref):
        dinv = _dinv_block(degp_ref)
        z_ref[...] = dinv * jnp.dot(x_ref[...], w_ref[...],
                                    preferred_element_type=jnp.float32)

    return pl.pallas_call(
        body,
        grid=(_N // _TCR,),
        in_specs=[
            pl.BlockSpec((2, _TCR, 128), lambda i: (0, i, 0)),
            pl.BlockSpec((_TCR, din), lambda i: (i, 0)),
            pl.BlockSpec((din, dout), lambda i: (0, 0)),
        ],
        out_specs=pl.BlockSpec((_TCR, dout), lambda i: (i, 0)),
        out_shape=jax.ShapeDtypeStruct((_N, dout), jnp.float32),
    )(degp, x, w0)


def _tc_mid(degp, u, zp, b, w):
    din, dout = w.shape

    def body(degp_ref, u_ref, zp_ref, b_ref, w_ref, z_ref):
        dinv = _dinv_block(degp_ref)
        h = jnp.maximum(dinv * (u_ref[0] + u_ref[1] + zp_ref[...]) + b_ref[...], 0.0)
        z_ref[...] = dinv * jnp.dot(h, w_ref[...],
                                    preferred_element_type=jnp.float32)

    return pl.pallas_call(
        body,
        grid=(_N // _TCR,),
        in_specs=[
            pl.BlockSpec((2, _TCR, 128), lambda i: (0, i, 0)),
            pl.BlockSpec((2, _TCR, din), lambda i: (0, i, 0)),
            pl.BlockSpec((_TCR, din), lambda i: (i, 0)),
            pl.BlockSpec((1, din), lambda i: (0, 0)),
            pl.BlockSpec((din, dout), lambda i: (0, 0)),
        ],
        out_specs=pl.BlockSpec((_TCR, dout), lambda i: (i, 0)),
        out_shape=jax.ShapeDtypeStruct((_N, dout), jnp.float32),
    )(degp, u, zp, b, w)


def _tc_last(degp, u, zp, b):
    dout = zp.shape[1]

    def body(degp_ref, u_ref, zp_ref, b_ref, o_ref):
        dinv = _dinv_block(degp_ref)
        o_ref[...] = jnp.maximum(
            dinv * (u_ref[0] + u_ref[1] + zp_ref[...]) + b_ref[...], 0.0)

    return pl.pallas_call(
        body,
        grid=(_N // _TCR,),
        in_specs=[
            pl.BlockSpec((2, _TCR, 128), lambda i: (0, i, 0)),
            pl.BlockSpec((2, _TCR, dout), lambda i: (0, i, 0)),
            pl.BlockSpec((_TCR, dout), lambda i: (i, 0)),
            pl.BlockSpec((1, dout), lambda i: (0, 0)),
        ],
        out_specs=pl.BlockSpec((_TCR, dout), lambda i: (i, 0)),
        out_shape=jax.ShapeDtypeStruct((_N, dout), jnp.float32),
    )(degp, u, zp, b)


def _pad_w(w, rows, cols):
    return jnp.pad(w, ((0, rows - w.shape[0]), (0, cols - w.shape[1])))


def _pad_b(b, cols):
    return jnp.pad(b, (0, cols - b.shape[0])).reshape(1, cols)


def kernel(x, edge_index, enc_W0, enc_b0, enc_W1, enc_b1,
           dec_W0, dec_b0, dec_W1, dec_b1):
    src = edge_index[0].reshape(_NTILES, 2, _NCH // 2, _CH)
    dst = edge_index[1].reshape(_NTILES, 2, _NCH // 2, _CH)

    w0 = _pad_w(enc_W0, 128, _DP)
    w1 = _pad_w(enc_W1, _DP, _DP)
    w2 = _pad_w(dec_W0, _DP, _DP)
    w3 = _pad_w(dec_W1, _DP, _DO)
    b0 = _pad_b(enc_b0, _DP)
    b1 = _pad_b(enc_b1, _DP)
    b2 = _pad_b(dec_b0, _DP)
    b3 = _pad_b(dec_b1, _DO)

    zeros_dp = jnp.zeros((_NPAD, _DP), jnp.float32)
    zeros_do = jnp.zeros((_NPAD, _DO), jnp.float32)
    ones128 = jnp.ones((_CH, 128), jnp.float32)

    degp = _make_deg_kernel()(dst, ones128, zeros_dp)

    edge_dp = _make_edge_kernel(_DP)
    edge_do = _make_edge_kernel(_DO)

    z0 = _tc_first(degp, x, w0)
    u0 = edge_dp(z0, src, dst, zeros_dp)
    z1 = _tc_mid(degp, u0, z0, b0, w1)
    u1 = edge_dp(z1, src, dst, zeros_dp)
    z2 = _tc_mid(degp, u1, z1, b1, w2)
    u2 = edge_dp(z2, src, dst, zeros_dp)
    z3 = _tc_mid(degp, u2, z2, b2, w3)
    u3 = edge_do(z3, src, dst, zeros_do)
    return _tc_last(degp, u3, z3, b3)


# speedup vs baseline: 25.5017x; 1.0185x over previous
import functools

import jax
import jax.numpy as jnp
from jax import lax
from jax.experimental import pallas as pl
from jax.experimental.pallas import tpu as pltpu
from jax.experimental.pallas import tpu_sc as plsc

_N = 10000
_E = 320000
_DP = 128
_DO = 128
_NTILES = 32
_EPT = _E // _NTILES
_CH = 125
_NCH = _EPT // _CH
_NPAD = 10240
_RPT = _NPAD // 16


def _flat_tile_id():
    c = lax.axis_index("c")
    s = lax.axis_index("s")
    return c, s, c * 16 + s


@functools.cache
def _make_edge_kernel(d):
    mesh = plsc.VectorSubcoreMesh(core_axis_name="c", subcore_axis_name="s")

    nbuf = 2
    hch = _NCH // 2

    @functools.partial(
        pl.kernel,
        out_type=jax.ShapeDtypeStruct((2, _NPAD, d), jnp.float32),
        mesh=mesh,
        scratch_types=[
            pltpu.VMEM((hch, _CH), jnp.int32),
            pltpu.VMEM((hch, _CH), jnp.int32),
            pltpu.VMEM((nbuf, _CH, d), jnp.float32),
            pltpu.VMEM_SHARED((_NPAD, d), jnp.float32),
            pltpu.SemaphoreType.DMA,
            pltpu.SemaphoreType.DMA,
        ],
    )
    def edge_kernel(z_hbm, src_hbm, dst_hbm, zero_hbm, out_hbm,
                    src_v, dst_v, rows_v, acc_sh, sem0, sem1):
        sems = (sem0, sem1)
        c, s, w = _flat_tile_id()
        row0 = pl.multiple_of(s * _RPT, 8)
        pltpu.sync_copy(zero_hbm.at[pl.ds(row0, _RPT)],
                        acc_sh.at[pl.ds(row0, _RPT)])
        plsc.subcore_barrier()

        def gather(j, b):
            pltpu.async_copy(z_hbm.at[src_v.at[j]], rows_v.at[b], sems[b])

        def gather_wait(j, b):
            pltpu.make_async_copy(z_hbm.at[src_v.at[j]], rows_v.at[b],
                                  sems[b]).wait()

        def scatter(j, b):
            pltpu.sync_copy(rows_v.at[b], acc_sh.at[dst_v.at[j]], add=True)

        for h in range(2):
            pltpu.sync_copy(src_hbm.at[w, h], src_v)
            pltpu.sync_copy(dst_hbm.at[w, h], dst_v)
            gather(0, 0)
            gather(1, 1)

            def body(t, carry):
                j = 2 * t
                gather_wait(j, 0)
                scatter(j, 0)

                @pl.when(j + 2 < hch)
                def _():
                    gather(j + 2, 0)

                gather_wait(j + 1, 1)
                scatter(j + 1, 1)

                @pl.when(j + 3 < hch)
                def _():
                    gather(j + 3, 1)

                return carry

            lax.fori_loop(0, hch // 2, body, 0)

        plsc.subcore_barrier()
        pltpu.sync_copy(acc_sh.at[pl.ds(row0, _RPT)],
                        out_hbm.at[c, pl.ds(row0, _RPT)])

    return edge_kernel


@functools.cache
def _make_deg_kernel():
    mesh = plsc.VectorSubcoreMesh(core_axis_name="c", subcore_axis_name="s")

    @functools.partial(
        pl.kernel,
        out_type=jax.ShapeDtypeStruct((2, _NPAD, 128), jnp.float32),
        mesh=mesh,
        scratch_types=[
            pltpu.VMEM((_NCH // 2, _CH), jnp.int32),
            pltpu.VMEM((_CH, 128), jnp.float32),
            pltpu.VMEM_SHARED((_NPAD, 128), jnp.float32),
        ],
    )
    def deg_kernel(dst_hbm, ones_hbm, zero_hbm, out_hbm, dst_v, ones_v, acc_sh):
        c, s, w = _flat_tile_id()
        row0 = pl.multiple_of(s * _RPT, 8)
        pltpu.sync_copy(ones_hbm, ones_v)
        pltpu.sync_copy(zero_hbm.at[pl.ds(row0, _RPT)],
                        acc_sh.at[pl.ds(row0, _RPT)])
        plsc.subcore_barrier()

        def body(j, carry):
            pltpu.sync_copy(ones_v, acc_sh.at[dst_v.at[j]], add=True)
            return carry

        for h in range(2):
            pltpu.sync_copy(dst_hbm.at[w, h], dst_v)
            lax.fori_loop(0, _NCH // 2, body, 0)
        plsc.subcore_barrier()
        pltpu.sync_copy(acc_sh.at[pl.ds(row0, _RPT)],
                        out_hbm.at[c, pl.ds(row0, _RPT)])

    return deg_kernel


_TCR = 2000


def _dinv_block(degp_ref):
    deg = degp_ref[0, :, 0:1] + degp_ref[1, :, 0:1] + 1.0
    return lax.rsqrt(deg)


def _tc_first(degp, x, w0):
    din, dout = x.shape[1], w0.shape[1]

    def body(degp_ref, x_ref, w_ref, z_ref):
        dinv = _dinv_block(degp_ref)
        z_ref[...] = dinv * jnp.dot(x_ref[...], w_ref[...],
                                    preferred_element_type=jnp.float32)

    return pl.pallas_call(
        body,
        grid=(_N // _TCR,),
        in_specs=[
            pl.BlockSpec((2, _TCR, 128), lambda i: (0, i, 0)),
            pl.BlockSpec((_TCR, din), lambda i: (i, 0)),
            pl.BlockSpec((din, dout), lambda i: (0, 0)),
        ],
        out_specs=pl.BlockSpec((_TCR, dout), lambda i: (i, 0)),
        out_shape=jax.ShapeDtypeStruct((_N, dout), jnp.float32),
    )(degp, x, w0)


def _tc_mid(degp, u, zp, b, w):
    din, dout = w.shape

    def body(degp_ref, u_ref, zp_ref, b_ref, w_ref, z_ref):
        dinv = _dinv_block(degp_ref)
        h = jnp.maximum(dinv * (u_ref[0] + u_ref[1] + zp_ref[...]) + b_ref[...], 0.0)
        z_ref[...] = dinv * jnp.dot(h, w_ref[...],
                                    preferred_element_type=jnp.float32)

    return pl.pallas_call(
        body,
        grid=(_N // _TCR,),
        in_specs=[
            pl.BlockSpec((2, _TCR, 128), lambda i: (0, i, 0)),
            pl.BlockSpec((2, _TCR, din), lambda i: (0, i, 0)),
            pl.BlockSpec((_TCR, din), lambda i: (i, 0)),
            pl.BlockSpec((1, din), lambda i: (0, 0)),
            pl.BlockSpec((din, dout), lambda i: (0, 0)),
        ],
        out_specs=pl.BlockSpec((_TCR, dout), lambda i: (i, 0)),
        out_shape=jax.ShapeDtypeStruct((_N, dout), jnp.float32),
    )(degp, u, zp, b, w)


def _tc_last(degp, u, zp, b):
    dout = zp.shape[1]

    def body(degp_ref, u_ref, zp_ref, b_ref, o_ref):
        dinv = _dinv_block(degp_ref)
        o_ref[...] = jnp.maximum(
            dinv * (u_ref[0] + u_ref[1] + zp_ref[...]) + b_ref[...], 0.0)

    return pl.pallas_call(
        body,
        grid=(_N // _TCR,),
        in_specs=[
            pl.BlockSpec((2, _TCR, 128), lambda i: (0, i, 0)),
            pl.BlockSpec((2, _TCR, dout), lambda i: (0, i, 0)),
            pl.BlockSpec((_TCR, dout), lambda i: (i, 0)),
            pl.BlockSpec((1, dout), lambda i: (0, 0)),
        ],
        out_specs=pl.BlockSpec((_TCR, dout), lambda i: (i, 0)),
        out_shape=jax.ShapeDtypeStruct((_N, dout), jnp.float32),
    )(degp, u, zp, b)


def _pad_w(w, rows, cols):
    return jnp.pad(w, ((0, rows - w.shape[0]), (0, cols - w.shape[1])))


def _pad_b(b, cols):
    return jnp.pad(b, (0, cols - b.shape[0])).reshape(1, cols)


def kernel(x, edge_index, enc_W0, enc_b0, enc_W1, enc_b1,
           dec_W0, dec_b0, dec_W1, dec_b1):
    src = edge_index[0].reshape(_NTILES, 2, _NCH // 2, _CH)
    dst = edge_index[1].reshape(_NTILES, 2, _NCH // 2, _CH)

    w0 = _pad_w(enc_W0, 128, _DP)
    w1 = _pad_w(enc_W1, _DP, _DP)
    w2 = _pad_w(dec_W0, _DP, _DP)
    w3 = _pad_w(dec_W1, _DP, _DO)
    b0 = _pad_b(enc_b0, _DP)
    b1 = _pad_b(enc_b1, _DP)
    b2 = _pad_b(dec_b0, _DP)
    b3 = _pad_b(dec_b1, _DO)

    zeros_dp = jnp.zeros((_NPAD, _DP), jnp.float32)
    zeros_do = jnp.zeros((_NPAD, _DO), jnp.float32)
    ones128 = jnp.ones((_CH, 128), jnp.float32)

    degp = _make_deg_kernel()(dst, ones128, zeros_dp)

    edge_dp = _make_edge_kernel(_DP)
    edge_do = _make_edge_kernel(_DO)

    z0 = _tc_first(degp, x, w0)
    u0 = edge_dp(z0, src, dst, zeros_dp)
    z1 = _tc_mid(degp, u0, z0, b0, w1)
    u1 = edge_dp(z1, src, dst, zeros_dp)
    z2 = _tc_mid(degp, u1, z1, b1, w2)
    u2 = edge_dp(z2, src, dst, zeros_dp)
    z3 = _tc_mid(degp, u2, z2, b2, w3)
    u3 = edge_do(z3, src, dst, zeros_do)
    return _tc_last(degp, u3, z3, b3)
